# baseline (speedup 1.0000x reference)
import jax
import jax.numpy as jnp
from jax.experimental import pallas as pl
from jax.experimental.pallas import tpu as pltpu

_N, _C, _H, _W = 4, 64, 496, 432
_HW = _H * _W
_CELLS = _N * _HW
_BLK = 3456
_NBLK = _CELLS // _BLK
_HBLK = _NBLK // _N


def _scatter_body(inter_ref, map_ref, out_ref):
    rows = inter_ref[...]
    m = map_ref[0, 0, :]
    cols = rows.T
    out_ref[0] = jnp.where((m >= 0)[None, :], cols, 0.0)


def _tc_scatter(inter, cell_map):
    map3 = cell_map.reshape(_NBLK, 1, _BLK)
    out = pl.pallas_call(
        _scatter_body,
        grid=(_NBLK,),
        in_specs=[
            pl.BlockSpec((_BLK, _C), lambda g: (g, 0)),
            pl.BlockSpec((1, 1, _BLK), lambda g: (g, 0, 0)),
        ],
        out_specs=pl.BlockSpec((1, _C, _BLK), lambda g: (g // _HBLK, 0, g % _HBLK)),
        out_shape=jax.ShapeDtypeStruct((_N, _C, _HW), jnp.float32),
    )(inter, map3)
    return out.reshape(_N, _C, _H, _W)


def kernel(voxel_features, coords):
    M = voxel_features.shape[0]
    flat = coords[:, 0] * _HW + coords[:, -2] * _W + coords[:, -1]
    cell_map = jnp.full((_CELLS,), -1, jnp.int32).at[flat].max(
        jnp.arange(M, dtype=jnp.int32))
    inter = voxel_features[jnp.clip(cell_map, 0), :]
    return _tc_scatter(inter, cell_map)


# speedup vs baseline: 1.9671x; 1.9671x over previous
import dataclasses
import functools

import jax
import jax.numpy as jnp
from jax import lax
from jax.experimental import pallas as pl
from jax.experimental.pallas import tpu as pltpu
from jax.experimental.pallas import tpu_sc as plsc

_N, _C, _H, _W = 4, 64, 496, 432
_HW = _H * _W
_CELLS = _N * _HW
_M = 40000
_NTILES = 32
_CPT = _CELLS // _NTILES
_NSEG = 2
_SEG = _CPT // _NSEG
_SEGGRP = _SEG // 16
_RCH = 256
_CAP = ((_SEG + _RCH - 1) // _RCH) * _RCH
_L = 16
_HUGE = 0x7FFFFFFF

_BLK = 3456
_NBLK = _CELLS // _BLK
_HBLK = _NBLK // _N


def _shift_up(v):
    idx = jnp.minimum(lax.iota(jnp.int32, _L) + 1, _L - 1)
    return lax.gather(
        v, idx[:, None],
        lax.GatherDimensionNumbers(offset_dims=(), collapsed_slice_dims=(0,),
                                   start_index_map=(0,)),
        (1,), mode=lax.GatherScatterMode.PROMISE_IN_BOUNDS)


def _sc_route(flat, feat):
    mesh = plsc.VectorSubcoreMesh(core_axis_name="c", subcore_axis_name="s")
    cp = pltpu.CompilerParams()
    if "needs_layout_passes" in pltpu.CompilerParams.__dataclass_fields__:
        cp = dataclasses.replace(cp, needs_layout_passes=False)
    if "use_tc_tiling_on_sc" in pltpu.CompilerParams.__dataclass_fields__:
        cp = dataclasses.replace(cp, use_tc_tiling_on_sc=False)

    @functools.partial(
        pl.kernel,
        mesh=mesh,
        compiler_params=cp,
        out_type=(
            jax.ShapeDtypeStruct((_CELLS,), jnp.int32),
            jax.ShapeDtypeStruct((_CELLS + _RCH, _C), jnp.float32),
        ),
        scratch_types=[
            pltpu.VMEM((_M,), jnp.int32),
            pltpu.VMEM((_CPT,), jnp.int32),
            pltpu.VMEM((_CAP,), jnp.int32),
            pltpu.VMEM((_CAP,), jnp.int32),
            pltpu.VMEM((_RCH, _C), jnp.float32),
            pltpu.VMEM((_RCH,), jnp.int32),
            pltpu.VMEM((_RCH,), jnp.int32),
        ],
    )
    def sc_kernel(flat_hbm, feat_hbm, map_hbm, inter_hbm,
                  flat_v, map_v, cid_v, mid_v, rows_v, stc_v, stm_v):
        lane = lax.iota(jnp.int32, _L)
        wid = lax.axis_index("s") * 2 + lax.axis_index("c")
        tbase = wid * _CPT

        pltpu.sync_copy(flat_hbm, flat_v)

        @pl.loop(0, _CPT, step=_L)
        def _(i):
            map_v[pl.ds(i, _L)] = jnp.broadcast_to(jnp.int32(-1), (_L,))

        def phase_a(g, carry):
            cell = flat_v[pl.ds(g * _L, _L)]
            inr = (cell >= tbase) & (cell < tbase + _CPT)
            loc = jnp.where(inr, cell - tbase, 0)
            key = jnp.where(inr, (loc << 4) | lane, _HUGE)
            ks, vs = plsc.sort_key_val(key, g * _L + lane)
            run_end = (lane == _L - 1) | ((ks >> 4) != (_shift_up(ks) >> 4))
            win = run_end & (ks != _HUGE)
            locw = jnp.where(win, ks >> 4, 0)
            plsc.store_scatter(map_v, [locw], vs, mask=win)
            return carry

        lax.fori_loop(0, _M // _L, phase_a, 0)

        for s in range(_NSEG):
            sbase = s * _SEG

            def scan(g, off, sbase=sbase):
                v = map_v[pl.ds(sbase + g * _L, _L)]
                occ = v >= 0
                gcell = tbase + sbase + g * _L + lane
                plsc.store_compressed(mid_v.at[pl.ds(off, _L)], v, mask=occ)
                plsc.store_compressed(cid_v.at[pl.ds(off, _L)], gcell, mask=occ)
                cnt = plsc.all_reduce_population_count(occ)
                return off + jnp.max(cnt)

            k = lax.fori_loop(0, _SEGGRP, scan, jnp.int32(0))
            kp = (k + _RCH - 1) & (~(_RCH - 1))

            for t in range(_RCH // _L):
                @pl.when(k + t * _L < kp)
                def _(t=t):
                    mid_v[pl.ds(k + t * _L, _L)] = jnp.broadcast_to(
                        jnp.int32(0), (_L,))
                    cid_v[pl.ds(k + t * _L, _L)] = _CELLS + t * _L + lane

            def chunk(j, carry):
                o = j * _RCH

                @pl.loop(0, _RCH, step=_L)
                def _(t):
                    stc_v[pl.ds(t, _L)] = cid_v[pl.ds(o + t, _L)]
                    stm_v[pl.ds(t, _L)] = mid_v[pl.ds(o + t, _L)]

                pltpu.sync_copy(feat_hbm.at[stm_v], rows_v)
                pltpu.sync_copy(rows_v, inter_hbm.at[stc_v])
                return carry

            lax.fori_loop(0, kp >> 8, chunk, 0)

        pltpu.sync_copy(map_v, map_hbm.at[pl.ds(tbase, _CPT)])

    return sc_kernel(flat, feat)


def _tc_body(inter_ref, map_ref, out_ref):
    rows = inter_ref[...]
    m = map_ref[0, 0, :]
    cols = rows.T
    out_ref[0] = jnp.where((m >= 0)[None, :], cols, 0.0)


def _tc_scatter(inter, cell_map):
    map3 = cell_map.reshape(_NBLK, 1, _BLK)
    out = pl.pallas_call(
        _tc_body,
        grid=(_NBLK,),
        in_specs=[
            pl.BlockSpec((_BLK, _C), lambda g: (g, 0)),
            pl.BlockSpec((1, 1, _BLK), lambda g: (g, 0, 0)),
        ],
        out_specs=pl.BlockSpec((1, _C, _BLK), lambda g: (g // _HBLK, 0, g % _HBLK)),
        out_shape=jax.ShapeDtypeStruct((_N, _C, _HW), jnp.float32),
    )(inter, map3)
    return out.reshape(_N, _C, _H, _W)


def kernel(voxel_features, coords):
    flat = coords[:, 0] * _HW + coords[:, -2] * _W + coords[:, -1]
    cell_map, inter = _sc_route(flat, voxel_features)
    return _tc_scatter(inter, cell_map)


# speedup vs baseline: 4.7201x; 2.3995x over previous
import dataclasses
import functools

import jax
import jax.numpy as jnp
from jax import lax
from jax.experimental import pallas as pl
from jax.experimental.pallas import tpu as pltpu
from jax.experimental.pallas import tpu_sc as plsc

_N, _C, _H, _W = 4, 64, 496, 432
_HW = _H * _W
_CELLS = _N * _HW
_M = 40000
_NTILES = 32
_CPT = _CELLS // _NTILES
_NSEG = 2
_SEG = _CPT // _NSEG
_SEGGRP = _SEG // 16
_RCH = 256
_CAP = ((_SEG + _RCH - 1) // _RCH) * _RCH
_L = 16
_HUGE = 0x7FFFFFFF

_XB = 8
_BLK = _XB * _H
_NBLK = _CELLS // _BLK
_WBLK = _NBLK // _N


def _shift_up(v):
    idx = jnp.minimum(lax.iota(jnp.int32, _L) + 1, _L - 1)
    return lax.gather(
        v, idx[:, None],
        lax.GatherDimensionNumbers(offset_dims=(), collapsed_slice_dims=(0,),
                                   start_index_map=(0,)),
        (1,), mode=lax.GatherScatterMode.PROMISE_IN_BOUNDS)


def _sc_route(flat, feat):
    mesh = plsc.VectorSubcoreMesh(core_axis_name="c", subcore_axis_name="s")
    cp = pltpu.CompilerParams()
    if "needs_layout_passes" in pltpu.CompilerParams.__dataclass_fields__:
        cp = dataclasses.replace(cp, needs_layout_passes=False)
    if "use_tc_tiling_on_sc" in pltpu.CompilerParams.__dataclass_fields__:
        cp = dataclasses.replace(cp, use_tc_tiling_on_sc=False)

    @functools.partial(
        pl.kernel,
        mesh=mesh,
        compiler_params=cp,
        out_type=(
            jax.ShapeDtypeStruct((_CELLS,), jnp.int32),
            jax.ShapeDtypeStruct((_CELLS + _RCH, _C), jnp.float32),
        ),
        scratch_types=[
            pltpu.VMEM((_M,), jnp.int32),
            pltpu.VMEM((_CPT,), jnp.int32),
            pltpu.VMEM((_CAP,), jnp.int32),
            pltpu.VMEM((_CAP,), jnp.int32),
            pltpu.VMEM((_RCH, _C), jnp.float32),
            pltpu.VMEM((_RCH,), jnp.int32),
            pltpu.VMEM((_RCH,), jnp.int32),
        ],
    )
    def sc_kernel(flat_hbm, feat_hbm, map_hbm, inter_hbm,
                  flat_v, map_v, cid_v, mid_v, rows_v, stc_v, stm_v):
        lane = lax.iota(jnp.int32, _L)
        wid = lax.axis_index("s") * 2 + lax.axis_index("c")
        tbase = wid * _CPT

        pltpu.sync_copy(flat_hbm, flat_v)

        @pl.loop(0, _CPT, step=_L)
        def _(i):
            map_v[pl.ds(i, _L)] = jnp.broadcast_to(jnp.int32(-1), (_L,))

        def phase_a(g, carry):
            cell = flat_v[pl.ds(g * _L, _L)]
            inr = (cell >= tbase) & (cell < tbase + _CPT)
            loc = jnp.where(inr, cell - tbase, 0)
            key = jnp.where(inr, (loc << 4) | lane, _HUGE)
            ks, vs = plsc.sort_key_val(key, g * _L + lane)
            run_end = (lane == _L - 1) | ((ks >> 4) != (_shift_up(ks) >> 4))
            win = run_end & (ks != _HUGE)
            locw = jnp.where(win, ks >> 4, 0)
            plsc.store_scatter(map_v, [locw], vs, mask=win)
            return carry

        lax.fori_loop(0, _M // _L, phase_a, 0)

        for s in range(_NSEG):
            sbase = s * _SEG

            def scan(g, off, sbase=sbase):
                v = map_v[pl.ds(sbase + g * _L, _L)]
                occ = v >= 0
                gcell = tbase + sbase + g * _L + lane
                plsc.store_compressed(mid_v.at[pl.ds(off, _L)], v, mask=occ)
                plsc.store_compressed(cid_v.at[pl.ds(off, _L)], gcell, mask=occ)
                cnt = plsc.all_reduce_population_count(occ)
                return off + jnp.max(cnt)

            k = lax.fori_loop(0, _SEGGRP, scan, jnp.int32(0))
            kp = (k + _RCH - 1) & (~(_RCH - 1))

            for t in range(_RCH // _L):
                @pl.when(k + t * _L < kp)
                def _(t=t):
                    mid_v[pl.ds(k + t * _L, _L)] = jnp.broadcast_to(
                        jnp.int32(0), (_L,))
                    cid_v[pl.ds(k + t * _L, _L)] = _CELLS + t * _L + lane

            def chunk(j, carry):
                o = j * _RCH

                @pl.loop(0, _RCH, step=_L)
                def _(t):
                    stc_v[pl.ds(t, _L)] = cid_v[pl.ds(o + t, _L)]
                    stm_v[pl.ds(t, _L)] = mid_v[pl.ds(o + t, _L)]

                pltpu.sync_copy(feat_hbm.at[stm_v], rows_v)
                pltpu.sync_copy(rows_v, inter_hbm.at[stc_v])
                return carry

            lax.fori_loop(0, kp >> 8, chunk, 0)

        pltpu.sync_copy(map_v, map_hbm.at[pl.ds(tbase, _CPT)])

    return sc_kernel(flat, feat)


def _tc_body(inter_ref, map_ref, out_ref):
    for j in range(_XB):
        t = inter_ref[j * _H:(j + 1) * _H, :].T
        mj = (map_ref[0, j, :] >= 0)[None, :]
        out_ref[0, :, j, :] = jnp.where(mj, t, 0.0)


def _tc_scatter(inter, cell_map):
    map3 = cell_map.reshape(_NBLK, _XB, _H)
    out = pl.pallas_call(
        _tc_body,
        grid=(_NBLK,),
        in_specs=[
            pl.BlockSpec((_BLK, _C), lambda g: (g, 0)),
            pl.BlockSpec((1, _XB, _H), lambda g: (g, 0, 0)),
        ],
        out_specs=pl.BlockSpec((1, _C, _XB, _H), lambda g: (g // _WBLK, 0, g % _WBLK, 0)),
        out_shape=jax.ShapeDtypeStruct((_N, _C, _W, _H), jnp.float32),
    )(inter, map3)
    return jnp.swapaxes(out, 2, 3)


def kernel(voxel_features, coords):
    flat = coords[:, 0] * _HW + coords[:, -1] * _H + coords[:, -2]
    cell_map, inter = _sc_route(flat, voxel_features)
    return _tc_scatter(inter, cell_map)


# speedup vs baseline: 5.3421x; 1.1318x over previous
import dataclasses
import functools

import jax
import jax.numpy as jnp
from jax import lax
from jax.experimental import pallas as pl
from jax.experimental.pallas import tpu as pltpu
from jax.experimental.pallas import tpu_sc as plsc

_N, _C, _H, _W = 4, 64, 496, 432
_HW = _H * _W
_CELLS = _N * _HW
_M = 40000
_NTILES = 32
_CPT = _CELLS // _NTILES
_NSEG = 2
_SEG = _CPT // _NSEG
_SEGGRP = _SEG // 16
_RCH = 256
_CW = 128
_CAP = ((_SEG + _RCH - 1) // _RCH) * _RCH
_L = 16
_HUGE = 0x7FFFFFFF

_XB = 8
_BLK = _XB * _H
_NBLK = _CELLS // _BLK
_WBLK = _NBLK // _N


def _shift_up(v):
    idx = jnp.minimum(lax.iota(jnp.int32, _L) + 1, _L - 1)
    return lax.gather(
        v, idx[:, None],
        lax.GatherDimensionNumbers(offset_dims=(), collapsed_slice_dims=(0,),
                                   start_index_map=(0,)),
        (1,), mode=lax.GatherScatterMode.PROMISE_IN_BOUNDS)


def _sc_route(flat, feat):
    mesh = plsc.VectorSubcoreMesh(core_axis_name="c", subcore_axis_name="s")
    cp = pltpu.CompilerParams()
    if "needs_layout_passes" in pltpu.CompilerParams.__dataclass_fields__:
        cp = dataclasses.replace(cp, needs_layout_passes=False)
    if "use_tc_tiling_on_sc" in pltpu.CompilerParams.__dataclass_fields__:
        cp = dataclasses.replace(cp, use_tc_tiling_on_sc=False)

    @functools.partial(
        pl.kernel,
        mesh=mesh,
        compiler_params=cp,
        out_type=(
            jax.ShapeDtypeStruct((_CELLS,), jnp.int32),
            jax.ShapeDtypeStruct((_CELLS + _RCH, _CW), jnp.float32),
        ),
        scratch_types=[
            pltpu.VMEM((_M,), jnp.int32),
            pltpu.VMEM((_CPT,), jnp.int32),
            pltpu.VMEM((_CAP,), jnp.int32),
            pltpu.VMEM((_CAP,), jnp.int32),
            pltpu.VMEM((_RCH, _CW), jnp.float32),
            pltpu.VMEM((_RCH,), jnp.int32),
            pltpu.VMEM((_RCH,), jnp.int32),
        ],
    )
    def sc_kernel(flat_hbm, feat_hbm, map_hbm, inter_hbm,
                  flat_v, map_v, cid_v, mid_v, rows_v, stc_v, stm_v):
        lane = lax.iota(jnp.int32, _L)
        wid = lax.axis_index("s") * 2 + lax.axis_index("c")
        tbase = wid * _CPT

        pltpu.sync_copy(flat_hbm, flat_v)

        @pl.loop(0, _CPT, step=_L)
        def _(i):
            map_v[pl.ds(i, _L)] = jnp.broadcast_to(jnp.int32(-1), (_L,))

        def phase_a(g, carry):
            cell = flat_v[pl.ds(g * _L, _L)]
            inr = (cell >= tbase) & (cell < tbase + _CPT)
            loc = jnp.where(inr, cell - tbase, 0)
            key = jnp.where(inr, (loc << 4) | lane, _HUGE)
            ks, vs = plsc.sort_key_val(key, g * _L + lane)
            run_end = (lane == _L - 1) | ((ks >> 4) != (_shift_up(ks) >> 4))
            win = run_end & (ks != _HUGE)
            locw = jnp.where(win, ks >> 4, 0)
            plsc.store_scatter(map_v, [locw], vs, mask=win)
            return carry

        lax.fori_loop(0, _M // _L, phase_a, 0)

        for s in range(_NSEG):
            sbase = s * _SEG

            def scan(g, off, sbase=sbase):
                v = map_v[pl.ds(sbase + g * _L, _L)]
                occ = v >= 0
                gcell = tbase + sbase + g * _L + lane
                plsc.store_compressed(mid_v.at[pl.ds(off, _L)], v, mask=occ)
                plsc.store_compressed(cid_v.at[pl.ds(off, _L)], gcell, mask=occ)
                cnt = plsc.all_reduce_population_count(occ)
                return off + jnp.max(cnt)

            k = lax.fori_loop(0, _SEGGRP, scan, jnp.int32(0))
            kp = (k + _RCH - 1) & (~(_RCH - 1))

            for t in range(_RCH // _L):
                @pl.when(k + t * _L < kp)
                def _(t=t):
                    mid_v[pl.ds(k + t * _L, _L)] = jnp.broadcast_to(
                        jnp.int32(0), (_L,))
                    cid_v[pl.ds(k + t * _L, _L)] = _CELLS + t * _L + lane

            def chunk(j, carry):
                o = j * _RCH

                @pl.loop(0, _RCH, step=_L)
                def _(t):
                    stc_v[pl.ds(t, _L)] = cid_v[pl.ds(o + t, _L)]
                    stm_v[pl.ds(t, _L)] = mid_v[pl.ds(o + t, _L)]

                pltpu.sync_copy(feat_hbm.at[stm_v], rows_v)
                pltpu.sync_copy(rows_v, inter_hbm.at[stc_v])
                return carry

            lax.fori_loop(0, kp >> 8, chunk, 0)

        pltpu.sync_copy(map_v, map_hbm.at[pl.ds(tbase, _CPT)])

    return sc_kernel(flat, feat)


def _tc_body(inter_ref, map_ref, out_ref):
    for j in range(_XB):
        t = inter_ref[j * _H:(j + 1) * _H, 0:_C].T
        mj = (map_ref[0, j, :] >= 0)[None, :]
        out_ref[0, :, j, :] = jnp.where(mj, t, 0.0)


def _tc_scatter(inter, cell_map):
    map3 = cell_map.reshape(_NBLK, _XB, _H)
    out = pl.pallas_call(
        _tc_body,
        grid=(_NBLK,),
        in_specs=[
            pl.BlockSpec((_BLK, _CW), lambda g: (g, 0)),
            pl.BlockSpec((1, _XB, _H), lambda g: (g, 0, 0)),
        ],
        out_specs=pl.BlockSpec((1, _C, _XB, _H), lambda g: (g // _WBLK, 0, g % _WBLK, 0)),
        out_shape=jax.ShapeDtypeStruct((_N, _C, _W, _H), jnp.float32),
    )(inter, map3)
    return jnp.swapaxes(out, 2, 3)


def kernel(voxel_features, coords):
    flat = coords[:, 0] * _HW + coords[:, -1] * _H + coords[:, -2]
    featp = jnp.pad(voxel_features, ((0, 0), (0, _CW - _C)))
    cell_map, inter = _sc_route(flat, featp)
    return _tc_scatter(inter, cell_map)


# speedup vs baseline: 5.8238x; 1.0902x over previous
import dataclasses
import functools

import jax
import jax.numpy as jnp
from jax import lax
from jax.experimental import pallas as pl
from jax.experimental.pallas import tpu as pltpu
from jax.experimental.pallas import tpu_sc as plsc

_N, _C, _H, _W = 4, 64, 496, 432
_HW = _H * _W
_CELLS = _N * _HW
_M = 40000
_NTILES = 32
_CPT = _CELLS // _NTILES
_NSEG = 2
_SEG = _CPT // _NSEG
_SEGGRP = _SEG // 16
_RCH = 256
_CW = 128
_CAP = ((_SEG + _RCH - 1) // _RCH) * _RCH
_L = 16
_HUGE = 0x7FFFFFFF

_XB = 16
_BLK = _XB * _H
_NBLK = _CELLS // _BLK
_WBLK = _NBLK // _N


def _shift_up(v):
    idx = jnp.minimum(lax.iota(jnp.int32, _L) + 1, _L - 1)
    return lax.gather(
        v, idx[:, None],
        lax.GatherDimensionNumbers(offset_dims=(), collapsed_slice_dims=(0,),
                                   start_index_map=(0,)),
        (1,), mode=lax.GatherScatterMode.PROMISE_IN_BOUNDS)


def _sc_route(flat, feat):
    mesh = plsc.VectorSubcoreMesh(core_axis_name="c", subcore_axis_name="s")
    cp = pltpu.CompilerParams()
    if "needs_layout_passes" in pltpu.CompilerParams.__dataclass_fields__:
        cp = dataclasses.replace(cp, needs_layout_passes=False)
    if "use_tc_tiling_on_sc" in pltpu.CompilerParams.__dataclass_fields__:
        cp = dataclasses.replace(cp, use_tc_tiling_on_sc=False)

    @functools.partial(
        pl.kernel,
        mesh=mesh,
        compiler_params=cp,
        out_type=(
            jax.ShapeDtypeStruct((_CELLS,), jnp.int32),
            jax.ShapeDtypeStruct((_CELLS + _RCH, _CW), jnp.float32),
        ),
        scratch_types=[
            pltpu.VMEM((_M,), jnp.int32),
            pltpu.VMEM((_CPT,), jnp.int32),
            pltpu.VMEM((_CAP,), jnp.int32),
            pltpu.VMEM((_CAP,), jnp.int32),
            pltpu.VMEM((_RCH, _CW), jnp.float32),
            pltpu.VMEM((_RCH,), jnp.int32),
            pltpu.VMEM((_RCH,), jnp.int32),
        ],
    )
    def sc_kernel(flat_hbm, feat_hbm, map_hbm, inter_hbm,
                  flat_v, map_v, cid_v, mid_v, rows_v, stc_v, stm_v):
        lane = lax.iota(jnp.int32, _L)
        wid = lax.axis_index("s") * 2 + lax.axis_index("c")
        tbase = wid * _CPT

        pltpu.sync_copy(flat_hbm, flat_v)

        @pl.loop(0, _CPT, step=_L)
        def _(i):
            map_v[pl.ds(i, _L)] = jnp.broadcast_to(jnp.int32(-1), (_L,))

        def phase_a(g, carry):
            cell = flat_v[pl.ds(g * _L, _L)]
            inr = (cell >= tbase) & (cell < tbase + _CPT)
            loc = jnp.where(inr, cell - tbase, 0)
            key = jnp.where(inr, (loc << 4) | lane, _HUGE)
            ks, vs = plsc.sort_key_val(key, g * _L + lane)
            run_end = (lane == _L - 1) | ((ks >> 4) != (_shift_up(ks) >> 4))
            win = run_end & (ks != _HUGE)
            locw = jnp.where(win, ks >> 4, 0)
            plsc.store_scatter(map_v, [locw], vs, mask=win)
            return carry

        lax.fori_loop(0, _M // _L, phase_a, 0)

        for s in range(_NSEG):
            sbase = s * _SEG

            def scan(g, off, sbase=sbase):
                v = map_v[pl.ds(sbase + g * _L, _L)]
                occ = v >= 0
                gcell = tbase + sbase + g * _L + lane
                plsc.store_compressed(mid_v.at[pl.ds(off, _L)], v, mask=occ)
                plsc.store_compressed(cid_v.at[pl.ds(off, _L)], gcell, mask=occ)
                cnt = plsc.all_reduce_population_count(occ)
                return off + jnp.max(cnt)

            k = lax.fori_loop(0, _SEGGRP, scan, jnp.int32(0))
            kp = (k + _RCH - 1) & (~(_RCH - 1))

            for t in range(_RCH // _L):
                @pl.when(k + t * _L < kp)
                def _(t=t):
                    mid_v[pl.ds(k + t * _L, _L)] = jnp.broadcast_to(
                        jnp.int32(0), (_L,))
                    cid_v[pl.ds(k + t * _L, _L)] = _CELLS + t * _L + lane

            def chunk(j, carry):
                o = j * _RCH

                @pl.loop(0, _RCH, step=_L)
                def _(t):
                    stc_v[pl.ds(t, _L)] = cid_v[pl.ds(o + t, _L)]
                    stm_v[pl.ds(t, _L)] = mid_v[pl.ds(o + t, _L)]

                pltpu.sync_copy(feat_hbm.at[stm_v], rows_v)
                pltpu.sync_copy(rows_v, inter_hbm.at[stc_v])
                return carry

            lax.fori_loop(0, kp >> 8, chunk, 0)

        pltpu.sync_copy(map_v, map_hbm.at[pl.ds(tbase, _CPT)])

    return sc_kernel(flat, feat)


def _tc_body(inter_ref, map_ref, out_ref):
    for j in range(_XB):
        t = inter_ref[j * _H:(j + 1) * _H, 0:_C].T
        mj = (map_ref[0, j, :] >= 0)[None, :]
        out_ref[0, :, j, :] = jnp.where(mj, t, 0.0)


def _tc_scatter(inter, cell_map):
    map3 = cell_map.reshape(_NBLK, _XB, _H)
    out = pl.pallas_call(
        _tc_body,
        grid=(_NBLK,),
        in_specs=[
            pl.BlockSpec((_BLK, _CW), lambda g: (g, 0)),
            pl.BlockSpec((1, _XB, _H), lambda g: (g, 0, 0)),
        ],
        out_specs=pl.BlockSpec((1, _C, _XB, _H), lambda g: (g // _WBLK, 0, g % _WBLK, 0)),
        out_shape=jax.ShapeDtypeStruct((_N, _C, _W, _H), jnp.float32),
    )(inter, map3)
    return jnp.swapaxes(out, 2, 3)


def kernel(voxel_features, coords):
    flat = coords[:, 0] * _HW + coords[:, -1] * _H + coords[:, -2]
    featp = jnp.pad(voxel_features, ((0, 0), (0, _CW - _C)))
    cell_map, inter = _sc_route(flat, featp)
    return _tc_scatter(inter, cell_map)


# speedup vs baseline: 5.9743x; 1.0258x over previous
import dataclasses
import functools

import jax
import jax.numpy as jnp
from jax import lax
from jax.experimental import pallas as pl
from jax.experimental.pallas import tpu as pltpu
from jax.experimental.pallas import tpu_sc as plsc

_N, _C, _H, _W = 4, 64, 496, 432
_HW = _H * _W
_CELLS = _N * _HW
_M = 40000
_NTILES = 32
_CPT = _CELLS // _NTILES
_NSEG = 2
_SEG = _CPT // _NSEG
_SEGGRP = _SEG // 16
_RCH = 256
_CW = 128
_CAP = ((_SEG + _RCH - 1) // _RCH) * _RCH
_L = 16
_HUGE = 0x7FFFFFFF

_XB = 48
_BLK = _XB * _H
_NBLK = _CELLS // _BLK
_WBLK = _NBLK // _N


def _shift_up(v):
    idx = jnp.minimum(lax.iota(jnp.int32, _L) + 1, _L - 1)
    return lax.gather(
        v, idx[:, None],
        lax.GatherDimensionNumbers(offset_dims=(), collapsed_slice_dims=(0,),
                                   start_index_map=(0,)),
        (1,), mode=lax.GatherScatterMode.PROMISE_IN_BOUNDS)


def _sc_route(flat, feat):
    mesh = plsc.VectorSubcoreMesh(core_axis_name="c", subcore_axis_name="s")
    cp = pltpu.CompilerParams()
    if "needs_layout_passes" in pltpu.CompilerParams.__dataclass_fields__:
        cp = dataclasses.replace(cp, needs_layout_passes=False)
    if "use_tc_tiling_on_sc" in pltpu.CompilerParams.__dataclass_fields__:
        cp = dataclasses.replace(cp, use_tc_tiling_on_sc=False)

    @functools.partial(
        pl.kernel,
        mesh=mesh,
        compiler_params=cp,
        out_type=(
            jax.ShapeDtypeStruct((_CELLS,), jnp.int32),
            jax.ShapeDtypeStruct((_CELLS + _RCH, _CW), jnp.float32),
        ),
        scratch_types=[
            pltpu.VMEM((_M,), jnp.int32),
            pltpu.VMEM((_CPT,), jnp.int32),
            pltpu.VMEM((_CAP,), jnp.int32),
            pltpu.VMEM((_CAP,), jnp.int32),
            pltpu.VMEM((_RCH, _CW), jnp.float32),
            pltpu.VMEM((_RCH,), jnp.int32),
            pltpu.VMEM((_RCH,), jnp.int32),
        ],
    )
    def sc_kernel(flat_hbm, feat_hbm, map_hbm, inter_hbm,
                  flat_v, map_v, cid_v, mid_v, rows_v, stc_v, stm_v):
        lane = lax.iota(jnp.int32, _L)
        wid = lax.axis_index("s") * 2 + lax.axis_index("c")
        tbase = wid * _CPT

        pltpu.sync_copy(flat_hbm, flat_v)

        @pl.loop(0, _CPT, step=_L)
        def _(i):
            map_v[pl.ds(i, _L)] = jnp.broadcast_to(jnp.int32(-1), (_L,))

        def phase_a(g, carry):
            cell = flat_v[pl.ds(g * _L, _L)]
            inr = (cell >= tbase) & (cell < tbase + _CPT)
            loc = jnp.where(inr, cell - tbase, 0)
            key = jnp.where(inr, (loc << 4) | lane, _HUGE)
            ks, vs = plsc.sort_key_val(key, g * _L + lane)
            run_end = (lane == _L - 1) | ((ks >> 4) != (_shift_up(ks) >> 4))
            win = run_end & (ks != _HUGE)
            locw = jnp.where(win, ks >> 4, 0)
            plsc.store_scatter(map_v, [locw], vs, mask=win)
            return carry

        lax.fori_loop(0, _M // _L, phase_a, 0)

        for s in range(_NSEG):
            sbase = s * _SEG

            def scan(g, off, sbase=sbase):
                v = map_v[pl.ds(sbase + g * _L, _L)]
                occ = v >= 0
                gcell = tbase + sbase + g * _L + lane
                plsc.store_compressed(mid_v.at[pl.ds(off, _L)], v, mask=occ)
                plsc.store_compressed(cid_v.at[pl.ds(off, _L)], gcell, mask=occ)
                cnt = plsc.all_reduce_population_count(occ)
                return off + jnp.max(cnt)

            k = lax.fori_loop(0, _SEGGRP, scan, jnp.int32(0))
            kp = (k + _RCH - 1) & (~(_RCH - 1))

            for t in range(_RCH // _L):
                @pl.when(k + t * _L < kp)
                def _(t=t):
                    mid_v[pl.ds(k + t * _L, _L)] = jnp.broadcast_to(
                        jnp.int32(0), (_L,))
                    cid_v[pl.ds(k + t * _L, _L)] = _CELLS + t * _L + lane

            def chunk(j, carry):
                o = j * _RCH

                @pl.loop(0, _RCH, step=_L)
                def _(t):
                    stc_v[pl.ds(t, _L)] = cid_v[pl.ds(o + t, _L)]
                    stm_v[pl.ds(t, _L)] = mid_v[pl.ds(o + t, _L)]

                pltpu.sync_copy(feat_hbm.at[stm_v], rows_v)
                pltpu.sync_copy(rows_v, inter_hbm.at[stc_v])
                return carry

            lax.fori_loop(0, kp >> 8, chunk, 0)

        pltpu.sync_copy(map_v, map_hbm.at[pl.ds(tbase, _CPT)])

    return sc_kernel(flat, feat)


_NQ = 8
_CG = _C // _NQ


def _tc_compute(inter_ref, map_ref, scr):
    for j in range(_XB):
        t = inter_ref[j * _H:(j + 1) * _H, 0:_C].T
        mj = (map_ref[0, j, :] >= 0)[None, :]
        scr[:, j, :] = jnp.where(mj, t, 0.0)


def _tc_copies(scr, sem, g, out_ref):
    n = g // _WBLK
    w0 = (g % _WBLK) * _XB
    return [
        pltpu.make_async_copy(
            scr.at[pl.ds(q * _CG, _CG)],
            out_ref.at[n, pl.ds(q * _CG, _CG), pl.ds(w0, _XB), :],
            sem)
        for q in range(_NQ)
    ]


def _tc_body(inter_ref, map_ref, out_ref, scr0, scr1, sem0, sem1):
    g = pl.program_id(0)
    even = (g % 2) == 0

    @pl.when(jnp.logical_and(even, g >= 2))
    def _():
        for c in _tc_copies(scr0, sem0, g - 2, out_ref):
            c.wait()

    @pl.when(jnp.logical_and(jnp.logical_not(even), g >= 2))
    def _():
        for c in _tc_copies(scr1, sem1, g - 2, out_ref):
            c.wait()

    @pl.when(even)
    def _():
        _tc_compute(inter_ref, map_ref, scr0)
        for c in _tc_copies(scr0, sem0, g, out_ref):
            c.start()

    @pl.when(jnp.logical_not(even))
    def _():
        _tc_compute(inter_ref, map_ref, scr1)
        for c in _tc_copies(scr1, sem1, g, out_ref):
            c.start()

    @pl.when(g == _NBLK - 1)
    def _():
        for c in _tc_copies(scr0, sem0, g - 1, out_ref):
            c.wait()
        for c in _tc_copies(scr1, sem1, g, out_ref):
            c.wait()


def _tc_scatter(inter, cell_map):
    map3 = cell_map.reshape(_NBLK, _XB, _H)
    out = pl.pallas_call(
        _tc_body,
        grid=(_NBLK,),
        in_specs=[
            pl.BlockSpec((_BLK, _CW), lambda g: (g, 0)),
            pl.BlockSpec((1, _XB, _H), lambda g: (g, 0, 0)),
        ],
        out_specs=pl.BlockSpec(memory_space=pl.ANY),
        out_shape=jax.ShapeDtypeStruct((_N, _C, _W, _H), jnp.float32),
        scratch_shapes=[
            pltpu.VMEM((_C, _XB, _H), jnp.float32),
            pltpu.VMEM((_C, _XB, _H), jnp.float32),
            pltpu.SemaphoreType.DMA,
            pltpu.SemaphoreType.DMA,
        ],
    )(inter, map3)
    return jnp.swapaxes(out, 2, 3)


def kernel(voxel_features, coords):
    flat = coords[:, 0] * _HW + coords[:, -1] * _H + coords[:, -2]
    featp = jnp.pad(voxel_features, ((0, 0), (0, _CW - _C)))
    cell_map, inter = _sc_route(flat, featp)
    return _tc_scatter(inter, cell_map)


# speedup vs baseline: 6.1894x; 1.0360x over previous
import dataclasses
import functools

import jax
import jax.numpy as jnp
from jax import lax
from jax.experimental import pallas as pl
from jax.experimental.pallas import tpu as pltpu
from jax.experimental.pallas import tpu_sc as plsc

_N, _C, _H, _W = 4, 64, 496, 432
_HW = _H * _W
_CELLS = _N * _HW
_M = 40000
_NTILES = 32
_CPT = _CELLS // _NTILES
_NSEG = 2
_SEG = _CPT // _NSEG
_SEGGRP = _SEG // 16
_RCH = 256
_CW = 128
_CAP = ((_SEG + _RCH - 1) // _RCH) * _RCH
_L = 16
_HUGE = 0x7FFFFFFF

_XB = 48
_BLK = _XB * _H
_NBLK = _CELLS // _BLK
_WBLK = _NBLK // _N


def _shift_up(v):
    idx = jnp.minimum(lax.iota(jnp.int32, _L) + 1, _L - 1)
    return lax.gather(
        v, idx[:, None],
        lax.GatherDimensionNumbers(offset_dims=(), collapsed_slice_dims=(0,),
                                   start_index_map=(0,)),
        (1,), mode=lax.GatherScatterMode.PROMISE_IN_BOUNDS)


def _sc_route(flat, feat):
    mesh = plsc.VectorSubcoreMesh(core_axis_name="c", subcore_axis_name="s")
    cp = pltpu.CompilerParams()
    if "needs_layout_passes" in pltpu.CompilerParams.__dataclass_fields__:
        cp = dataclasses.replace(cp, needs_layout_passes=False)
    if "use_tc_tiling_on_sc" in pltpu.CompilerParams.__dataclass_fields__:
        cp = dataclasses.replace(cp, use_tc_tiling_on_sc=False)

    @functools.partial(
        pl.kernel,
        mesh=mesh,
        compiler_params=cp,
        out_type=(
            jax.ShapeDtypeStruct((_CELLS,), jnp.int32),
            jax.ShapeDtypeStruct((_CELLS + _RCH, _CW), jnp.float32),
        ),
        scratch_types=[
            pltpu.VMEM((_M,), jnp.int32),
            pltpu.VMEM((_CPT,), jnp.int32),
            pltpu.VMEM((_CAP,), jnp.int32),
            pltpu.VMEM((_CAP,), jnp.int32),
            pltpu.VMEM((_RCH, _CW), jnp.float32),
            pltpu.VMEM((_RCH,), jnp.int32),
            pltpu.VMEM((_RCH,), jnp.int32),
        ],
    )
    def sc_kernel(flat_hbm, feat_hbm, map_hbm, inter_hbm,
                  flat_v, map_v, cid_v, mid_v, rows_v, stc_v, stm_v):
        lane = lax.iota(jnp.int32, _L)
        wid = lax.axis_index("s") * 2 + lax.axis_index("c")
        tbase = wid * _CPT

        pltpu.sync_copy(flat_hbm, flat_v)

        @pl.loop(0, _CPT, step=_L)
        def _(i):
            map_v[pl.ds(i, _L)] = jnp.broadcast_to(jnp.int32(-1), (_L,))

        def winners(m0):
            cell = flat_v[pl.ds(m0, _L)]
            inr = (cell >= tbase) & (cell < tbase + _CPT)
            loc = jnp.where(inr, cell - tbase, 0)
            key = jnp.where(inr, (loc << 4) | lane, _HUGE)
            ks, vs = plsc.sort_key_val(key, m0 + lane)
            run_end = (lane == _L - 1) | ((ks >> 4) != (_shift_up(ks) >> 4))
            win = run_end & (ks != _HUGE)
            return jnp.where(win, ks >> 4, 0), vs, win

        def phase_a(g, carry):
            l0, v0, w0 = winners(g * 2 * _L)
            l1, v1, w1 = winners(g * 2 * _L + _L)
            plsc.store_scatter(map_v, [l0], v0, mask=w0)
            plsc.store_scatter(map_v, [l1], v1, mask=w1)
            return carry

        lax.fori_loop(0, _M // (2 * _L), phase_a, 0)

        for s in range(_NSEG):
            sbase = s * _SEG

            def scan(g, off, sbase=sbase):
                v = map_v[pl.ds(sbase + g * _L, _L)]
                occ = v >= 0
                gcell = tbase + sbase + g * _L + lane
                plsc.store_compressed(mid_v.at[pl.ds(off, _L)], v, mask=occ)
                plsc.store_compressed(cid_v.at[pl.ds(off, _L)], gcell, mask=occ)
                cnt = plsc.all_reduce_population_count(occ)
                return off + jnp.max(cnt)

            k = lax.fori_loop(0, _SEGGRP, scan, jnp.int32(0))
            kp = (k + _RCH - 1) & (~(_RCH - 1))

            for t in range(_RCH // _L):
                @pl.when(k + t * _L < kp)
                def _(t=t):
                    mid_v[pl.ds(k + t * _L, _L)] = jnp.broadcast_to(
                        jnp.int32(0), (_L,))
                    cid_v[pl.ds(k + t * _L, _L)] = _CELLS + t * _L + lane

            def chunk(j, carry):
                o = j * _RCH

                @pl.loop(0, _RCH, step=_L)
                def _(t):
                    stc_v[pl.ds(t, _L)] = cid_v[pl.ds(o + t, _L)]
                    stm_v[pl.ds(t, _L)] = mid_v[pl.ds(o + t, _L)]

                pltpu.sync_copy(feat_hbm.at[stm_v], rows_v)
                pltpu.sync_copy(rows_v, inter_hbm.at[stc_v])
                return carry

            lax.fori_loop(0, kp >> 8, chunk, 0)

        pltpu.sync_copy(map_v, map_hbm.at[pl.ds(tbase, _CPT)])

    return sc_kernel(flat, feat)


_NQ = 8
_CG = _C // _NQ


def _tc_compute(inter_ref, map_ref, scr):
    for j in range(_XB):
        t = inter_ref[j * _H:(j + 1) * _H, 0:_C].T
        mj = (map_ref[0, j, :] >= 0)[None, :]
        scr[:, j, :] = jnp.where(mj, t, 0.0)


def _tc_copies(scr, sem, g, out_ref):
    n = g // _WBLK
    w0 = (g % _WBLK) * _XB
    return [
        pltpu.make_async_copy(
            scr.at[pl.ds(q * _CG, _CG)],
            out_ref.at[n, pl.ds(q * _CG, _CG), pl.ds(w0, _XB), :],
            sem)
        for q in range(_NQ)
    ]


def _tc_body(inter_ref, map_ref, out_ref, scr0, scr1, sem0, sem1):
    g = pl.program_id(0)
    even = (g % 2) == 0

    @pl.when(jnp.logical_and(even, g >= 2))
    def _():
        for c in _tc_copies(scr0, sem0, g - 2, out_ref):
            c.wait()

    @pl.when(jnp.logical_and(jnp.logical_not(even), g >= 2))
    def _():
        for c in _tc_copies(scr1, sem1, g - 2, out_ref):
            c.wait()

    @pl.when(even)
    def _():
        _tc_compute(inter_ref, map_ref, scr0)
        for c in _tc_copies(scr0, sem0, g, out_ref):
            c.start()

    @pl.when(jnp.logical_not(even))
    def _():
        _tc_compute(inter_ref, map_ref, scr1)
        for c in _tc_copies(scr1, sem1, g, out_ref):
            c.start()

    @pl.when(g == _NBLK - 1)
    def _():
        for c in _tc_copies(scr0, sem0, g - 1, out_ref):
            c.wait()
        for c in _tc_copies(scr1, sem1, g, out_ref):
            c.wait()


def _tc_scatter(inter, cell_map):
    map3 = cell_map.reshape(_NBLK, _XB, _H)
    out = pl.pallas_call(
        _tc_body,
        grid=(_NBLK,),
        in_specs=[
            pl.BlockSpec((_BLK, _CW), lambda g: (g, 0)),
            pl.BlockSpec((1, _XB, _H), lambda g: (g, 0, 0)),
        ],
        out_specs=pl.BlockSpec(memory_space=pl.ANY),
        out_shape=jax.ShapeDtypeStruct((_N, _C, _W, _H), jnp.float32),
        scratch_shapes=[
            pltpu.VMEM((_C, _XB, _H), jnp.float32),
            pltpu.VMEM((_C, _XB, _H), jnp.float32),
            pltpu.SemaphoreType.DMA,
            pltpu.SemaphoreType.DMA,
        ],
    )(inter, map3)
    return jnp.swapaxes(out, 2, 3)


def kernel(voxel_features, coords):
    flat = coords[:, 0] * _HW + coords[:, -1] * _H + coords[:, -2]
    featp = jnp.pad(voxel_features, ((0, 0), (0, _CW - _C)))
    cell_map, inter = _sc_route(flat, featp)
    return _tc_scatter(inter, cell_map)


# speedup vs baseline: 6.2942x; 1.0169x over previous
import dataclasses
import functools

import jax
import jax.numpy as jnp
from jax import lax
from jax.experimental import pallas as pl
from jax.experimental.pallas import tpu as pltpu
from jax.experimental.pallas import tpu_sc as plsc

_N, _C, _H, _W = 4, 64, 496, 432
_HW = _H * _W
_CELLS = _N * _HW
_M = 40000
_NTILES = 32
_CPT = _CELLS // _NTILES
_NSEG = 2
_SEG = _CPT // _NSEG
_SEGGRP = _SEG // 16
_RCH = 256
_CW = 128
_CAP = ((_SEG + _RCH - 1) // _RCH) * _RCH
_L = 16
_HUGE = 0x7FFFFFFF

_XB = 48
_BLK = _XB * _H
_NBLK = _CELLS // _BLK
_WBLK = _NBLK // _N


def _shift_up(v):
    idx = jnp.minimum(lax.iota(jnp.int32, _L) + 1, _L - 1)
    return lax.gather(
        v, idx[:, None],
        lax.GatherDimensionNumbers(offset_dims=(), collapsed_slice_dims=(0,),
                                   start_index_map=(0,)),
        (1,), mode=lax.GatherScatterMode.PROMISE_IN_BOUNDS)


def _sc_route(flat, feat):
    mesh = plsc.VectorSubcoreMesh(core_axis_name="c", subcore_axis_name="s")
    cp = pltpu.CompilerParams()
    if "needs_layout_passes" in pltpu.CompilerParams.__dataclass_fields__:
        cp = dataclasses.replace(cp, needs_layout_passes=False)
    if "use_tc_tiling_on_sc" in pltpu.CompilerParams.__dataclass_fields__:
        cp = dataclasses.replace(cp, use_tc_tiling_on_sc=False)

    @functools.partial(
        pl.kernel,
        mesh=mesh,
        compiler_params=cp,
        out_type=(
            jax.ShapeDtypeStruct((_CELLS,), jnp.int32),
            jax.ShapeDtypeStruct((_CELLS + _RCH, _CW), jnp.float32),
        ),
        scratch_types=[
            pltpu.VMEM((_M,), jnp.int32),
            pltpu.VMEM((_CPT,), jnp.int32),
            pltpu.VMEM((_CAP,), jnp.int32),
            pltpu.VMEM((_CAP,), jnp.int32),
            pltpu.VMEM((_RCH, _CW), jnp.float32),
            pltpu.VMEM((_RCH,), jnp.int32),
            pltpu.VMEM((_RCH,), jnp.int32),
        ],
    )
    def sc_kernel(flat_hbm, feat_hbm, map_hbm, inter_hbm,
                  flat_v, map_v, cid_v, mid_v, rows_v, stc_v, stm_v):
        lane = lax.iota(jnp.int32, _L)
        wid = lax.axis_index("s") * 2 + lax.axis_index("c")
        tbase = wid * _CPT

        pltpu.sync_copy(flat_hbm, flat_v)

        @pl.loop(0, _CPT, step=_L)
        def _(i):
            map_v[pl.ds(i, _L)] = jnp.broadcast_to(jnp.int32(-1), (_L,))

        def winners(m0):
            cell = flat_v[pl.ds(m0, _L)]
            inr = (cell >= tbase) & (cell < tbase + _CPT)
            loc = jnp.where(inr, cell - tbase, 0)
            key = jnp.where(inr, (loc << 4) | lane, _HUGE)
            ks, vs = plsc.sort_key_val(key, m0 + lane)
            run_end = (lane == _L - 1) | ((ks >> 4) != (_shift_up(ks) >> 4))
            win = run_end & (ks != _HUGE)
            return jnp.where(win, ks >> 4, 0), vs, win

        def phase_a(g, carry):
            parts = [winners(g * 4 * _L + h * _L) for h in range(4)]
            for l, v, w in parts:
                plsc.store_scatter(map_v, [l], v, mask=w)
            return carry

        lax.fori_loop(0, _M // (4 * _L), phase_a, 0)

        for s in range(_NSEG):
            sbase = s * _SEG

            def scan(g, off, sbase=sbase):
                v = map_v[pl.ds(sbase + g * _L, _L)]
                occ = v >= 0
                gcell = tbase + sbase + g * _L + lane
                plsc.store_compressed(mid_v.at[pl.ds(off, _L)], v, mask=occ)
                plsc.store_compressed(cid_v.at[pl.ds(off, _L)], gcell, mask=occ)
                cnt = plsc.all_reduce_population_count(occ)
                return off + jnp.max(cnt)

            k = lax.fori_loop(0, _SEGGRP, scan, jnp.int32(0))
            kp = (k + _RCH - 1) & (~(_RCH - 1))

            for t in range(_RCH // _L):
                @pl.when(k + t * _L < kp)
                def _(t=t):
                    mid_v[pl.ds(k + t * _L, _L)] = jnp.broadcast_to(
                        jnp.int32(0), (_L,))
                    cid_v[pl.ds(k + t * _L, _L)] = _CELLS + t * _L + lane

            def chunk(j, carry):
                o = j * _RCH

                @pl.loop(0, _RCH, step=_L)
                def _(t):
                    stc_v[pl.ds(t, _L)] = cid_v[pl.ds(o + t, _L)]
                    stm_v[pl.ds(t, _L)] = mid_v[pl.ds(o + t, _L)]

                pltpu.sync_copy(feat_hbm.at[stm_v], rows_v)
                pltpu.sync_copy(rows_v, inter_hbm.at[stc_v])
                return carry

            lax.fori_loop(0, kp >> 8, chunk, 0)

        pltpu.sync_copy(map_v, map_hbm.at[pl.ds(tbase, _CPT)])

    return sc_kernel(flat, feat)


_NQ = 8
_CG = _C // _NQ


def _tc_compute(inter_ref, map_ref, scr):
    for j in range(_XB):
        t = inter_ref[j * _H:(j + 1) * _H, 0:_C].T
        mj = (map_ref[0, j, :] >= 0)[None, :]
        scr[:, j, :] = jnp.where(mj, t, 0.0)


def _tc_copies(scr, sem, g, out_ref):
    n = g // _WBLK
    w0 = (g % _WBLK) * _XB
    return [
        pltpu.make_async_copy(
            scr.at[pl.ds(q * _CG, _CG)],
            out_ref.at[n, pl.ds(q * _CG, _CG), pl.ds(w0, _XB), :],
            sem)
        for q in range(_NQ)
    ]


def _tc_body(inter_ref, map_ref, out_ref, scr0, scr1, sem0, sem1):
    g = pl.program_id(0)
    even = (g % 2) == 0

    @pl.when(jnp.logical_and(even, g >= 2))
    def _():
        for c in _tc_copies(scr0, sem0, g - 2, out_ref):
            c.wait()

    @pl.when(jnp.logical_and(jnp.logical_not(even), g >= 2))
    def _():
        for c in _tc_copies(scr1, sem1, g - 2, out_ref):
            c.wait()

    @pl.when(even)
    def _():
        _tc_compute(inter_ref, map_ref, scr0)
        for c in _tc_copies(scr0, sem0, g, out_ref):
            c.start()

    @pl.when(jnp.logical_not(even))
    def _():
        _tc_compute(inter_ref, map_ref, scr1)
        for c in _tc_copies(scr1, sem1, g, out_ref):
            c.start()

    @pl.when(g == _NBLK - 1)
    def _():
        for c in _tc_copies(scr0, sem0, g - 1, out_ref):
            c.wait()
        for c in _tc_copies(scr1, sem1, g, out_ref):
            c.wait()


def _tc_scatter(inter, cell_map):
    map3 = cell_map.reshape(_NBLK, _XB, _H)
    out = pl.pallas_call(
        _tc_body,
        grid=(_NBLK,),
        in_specs=[
            pl.BlockSpec((_BLK, _CW), lambda g: (g, 0)),
            pl.BlockSpec((1, _XB, _H), lambda g: (g, 0, 0)),
        ],
        out_specs=pl.BlockSpec(memory_space=pl.ANY),
        out_shape=jax.ShapeDtypeStruct((_N, _C, _W, _H), jnp.float32),
        scratch_shapes=[
            pltpu.VMEM((_C, _XB, _H), jnp.float32),
            pltpu.VMEM((_C, _XB, _H), jnp.float32),
            pltpu.SemaphoreType.DMA,
            pltpu.SemaphoreType.DMA,
        ],
    )(inter, map3)
    return jnp.swapaxes(out, 2, 3)


def kernel(voxel_features, coords):
    flat = coords[:, 0] * _HW + coords[:, -1] * _H + coords[:, -2]
    featp = jnp.pad(voxel_features, ((0, 0), (0, _CW - _C)))
    cell_map, inter = _sc_route(flat, featp)
    return _tc_scatter(inter, cell_map)
